# baseline (speedup 1.0000x reference)
import functools

import jax
import jax.numpy as jnp
from jax import lax
from jax.experimental import pallas as pl
from jax.experimental.pallas import tpu as pltpu
from jax.experimental.pallas import tpu_sc as plsc

N = 10000
E = 160000
NPAD = 10240
TRASH = N
NC, NS = 2, 16
CHUNK = 128
SEC = 8
B = 10000


def _sc_segsum(C, D):
    mesh = plsc.VectorSubcoreMesh(core_axis_name="c", subcore_axis_name="s",
                                   num_cores=NC, num_subcores=NS)
    zrows = NPAD // NS
    nsec = C // SEC
    psec = SEC // 2
    assert C % SEC == 0

    @functools.partial(
        pl.kernel,
        out_type=jax.ShapeDtypeStruct((2, NPAD, D), jnp.float32),
        mesh=mesh,
        scratch_types=[
            pltpu.VMEM((2, SEC, CHUNK), jnp.int32),
            pltpu.VMEM((2, SEC, CHUNK), jnp.int32),
            pltpu.VMEM((CHUNK, D), jnp.float32),
            pltpu.VMEM((CHUNK, D), jnp.float32),
            pltpu.VMEM((16, D), jnp.float32),
            pltpu.VMEM_SHARED((NPAD, D), jnp.float32),
            pltpu.SemaphoreType.DMA,
            pltpu.SemaphoreType.DMA,
            pltpu.SemaphoreType.DMA,
            pltpu.SemaphoreType.DMA,
            pltpu.SemaphoreType.DMA,
        ],
    )
    def k(table, src_i, dst_i, out, src_s, dst_s, g0, g1, zbuf, acc,
          sg0, sg1, ss0, ss1, sz):
        c = lax.axis_index("c")
        s = lax.axis_index("s")

        for r in range(16):
            for q in range(D // 16):
                zbuf[r, pl.ds(q * 16, 16)] = jnp.zeros((16,), jnp.float32)

        def zbody(i, carry):
            pltpu.async_copy(zbuf, acc.at[pl.ds(s * zrows + i * 16, 16)], sz)
            return carry
        lax.fori_loop(0, zrows // 16, zbody, 0)
        pltpu.sync_copy(src_i.at[c, s, pl.ds(0, SEC)], src_s.at[0])
        pltpu.sync_copy(dst_i.at[c, s, pl.ds(0, SEC)], dst_s.at[0])

        def zdrain(i, carry):
            pltpu.make_async_copy(zbuf, acc.at[pl.ds(s * zrows, 16)], sz).wait()
            return carry
        lax.fori_loop(0, zrows // 16, zdrain, 0)
        plsc.subcore_barrier()

        def src_sl(x):
            return src_s.at[(x // SEC) % 2, x % SEC]

        def dst_sl(x):
            return dst_s.at[(x // SEC) % 2, x % SEC]

        def wait_g(buf, sem, x):
            pltpu.make_async_copy(table.at[src_sl(x)], buf, sem).wait()

        def wait_s(buf, sem, x):
            pltpu.make_async_copy(buf, acc.at[dst_sl(x)], sem).wait()

        pltpu.async_copy(table.at[src_sl(0)], g0, sg0)

        def mbody(i, carry):
            a = 2 * i
            b = a + 1
            sec = i // psec
            wait_g(g0, sg0, a)
            pltpu.async_copy(g0, acc.at[dst_sl(a)], ss0, add=True)

            @pl.when(i > 0)
            def _():
                wait_s(g1, ss1, a)

            @pl.when((i % psec == 0) & (sec + 1 < nsec))
            def _():
                pltpu.async_copy(src_i.at[c, s, pl.ds((sec + 1) * SEC, SEC)],
                                 src_s.at[(sec + 1) % 2], sz)
                pltpu.async_copy(dst_i.at[c, s, pl.ds((sec + 1) * SEC, SEC)],
                                 dst_s.at[(sec + 1) % 2], sz)

            pltpu.async_copy(table.at[src_sl(b)], g1, sg1)
            wait_g(g1, sg1, b)
            pltpu.async_copy(g1, acc.at[dst_sl(b)], ss1, add=True)
            wait_s(g0, ss0, a)

            @pl.when(((i + 1) % psec == 0) & (sec + 1 < nsec))
            def _():
                pltpu.make_async_copy(src_i.at[c, s, pl.ds(0, SEC)],
                                      src_s.at[0], sz).wait()
                pltpu.make_async_copy(dst_i.at[c, s, pl.ds(0, SEC)],
                                      dst_s.at[0], sz).wait()

            @pl.when(b + 1 < C)
            def _():
                pltpu.async_copy(table.at[src_sl(b + 1)], g0, sg0)
            return carry
        lax.fori_loop(0, C // 2, mbody, 0)

        wait_s(g1, ss1, C - 1)
        plsc.subcore_barrier()
        pltpu.sync_copy(acc.at[pl.ds(s * zrows, zrows)],
                        out.at[c, pl.ds(s * zrows, zrows)])

    return k


def _sc_counts(C):
    D = 128
    mesh = plsc.VectorSubcoreMesh(core_axis_name="c", subcore_axis_name="s",
                                   num_cores=NC, num_subcores=NS)
    zrows = NPAD // NS

    @functools.partial(
        pl.kernel,
        out_type=jax.ShapeDtypeStruct((2, NPAD, D), jnp.float32),
        mesh=mesh,
        scratch_types=[
            pltpu.VMEM((C, CHUNK), jnp.int32),
            pltpu.VMEM((CHUNK, D), jnp.float32),
            pltpu.VMEM((16, D), jnp.float32),
            pltpu.VMEM_SHARED((NPAD, D), jnp.float32),
            pltpu.SemaphoreType.DMA,
        ],
    )
    def k(dst_i, out, dst_v, ones_b, zbuf, acc, sz):
        c = lax.axis_index("c")
        s = lax.axis_index("s")

        for r in range(16):
            for q in range(D // 16):
                zbuf[r, pl.ds(q * 16, 16)] = jnp.zeros((16,), jnp.float32)
        for r in range(CHUNK):
            for q in range(D // 16):
                ones_b[r, pl.ds(q * 16, 16)] = jnp.ones((16,), jnp.float32)

        def zbody(i, carry):
            pltpu.async_copy(zbuf, acc.at[pl.ds(s * zrows + i * 16, 16)], sz)
            return carry
        lax.fori_loop(0, zrows // 16, zbody, 0)

        pltpu.sync_copy(dst_i.at[c, s], dst_v)

        def zdrain(i, carry):
            pltpu.make_async_copy(zbuf, acc.at[pl.ds(s * zrows, 16)], sz).wait()
            return carry
        lax.fori_loop(0, zrows // 16, zdrain, 0)
        plsc.subcore_barrier()

        def mbody(j, carry):
            pltpu.sync_copy(ones_b, acc.at[dst_v.at[j]], add=True)
            return carry
        lax.fori_loop(0, C, mbody, 0)

        plsc.subcore_barrier()
        pltpu.sync_copy(acc.at[pl.ds(s * zrows, zrows)],
                        out.at[c, pl.ds(s * zrows, zrows)])

    return k


def _tc_pre_body(x_ref, wl_ref, y_ref):
    y = x_ref[...] @ wl_ref[...]
    y_ref[0] = y[:, :128]
    y_ref[1] = y[:, 128:]


def _tc_pre(x, wl):
    return pl.pallas_call(
        _tc_pre_body,
        grid=(N // B,),
        in_specs=[
            pl.BlockSpec((B, 256), lambda b: (b, 0)),
            pl.BlockSpec((256, 256), lambda b: (0, 0)),
        ],
        out_specs=pl.BlockSpec((2, B, 128), lambda b: (0, b, 0)),
        out_shape=jax.ShapeDtypeStruct((2, N, 128), jnp.float32),
    )(x, wl)


def _mean_root(s_ref, h_ref, cnt_ref, bias_ref, wr_ref, split):
    cnt = cnt_ref[0, :, 0:1] + cnt_ref[1, :, 0:1]
    inv = 1.0 / jnp.maximum(cnt, 1.0)
    if split:
        ssum = jnp.concatenate([s_ref[0], s_ref[1]], axis=1)
    else:
        ssum = s_ref[0] + s_ref[1]
    pre = ssum * inv + bias_ref[...][None, :] + h_ref[...] @ wr_ref[...]
    return jnp.maximum(pre, 0.0)


def _tc_mid_body(split_next, s_ref, h_ref, cnt_ref, bias_ref, wr_ref, wln_ref,
                 hout_ref, y_ref):
    h = _mean_root(s_ref, h_ref, cnt_ref, bias_ref, wr_ref, split=True)
    hout_ref[...] = h
    y = h @ wln_ref[...]
    if split_next:
        y_ref[0] = y[:, :128]
        y_ref[1] = y[:, 128:]
    else:
        y_ref[...] = y


def _tc_mid(s, h_prev, cnt_pad, bias, wr, wl_next, split_next):
    d_next = wl_next.shape[1]
    if split_next:
        y_shape = jax.ShapeDtypeStruct((2, N, 128), jnp.float32)
        y_spec = pl.BlockSpec((2, B, 128), lambda b: (0, b, 0))
    else:
        y_shape = jax.ShapeDtypeStruct((N, d_next), jnp.float32)
        y_spec = pl.BlockSpec((B, d_next), lambda b: (b, 0))
    return pl.pallas_call(
        functools.partial(_tc_mid_body, split_next),
        grid=(N // B,),
        in_specs=[
            pl.BlockSpec((2, B, 128), lambda b: (0, b, 0)),
            pl.BlockSpec((B, 256), lambda b: (b, 0)),
            pl.BlockSpec((2, B, 128), lambda b: (0, b, 0)),
            pl.BlockSpec((256,), lambda b: (0,)),
            pl.BlockSpec((256, 256), lambda b: (0, 0)),
            pl.BlockSpec((256, d_next), lambda b: (0, 0)),
        ],
        out_specs=[
            pl.BlockSpec((B, 256), lambda b: (b, 0)),
            y_spec,
        ],
        out_shape=[
            jax.ShapeDtypeStruct((N, 256), jnp.float32),
            y_shape,
        ],
    )(s, h_prev, cnt_pad, bias, wr, wl_next)


def _tc_fin_body(s_ref, h_ref, cnt_ref, bias_ref, wr_ref, wo_ref, bo_ref,
                 o_ref):
    h = _mean_root(s_ref, h_ref, cnt_ref, bias_ref, wr_ref, split=False)
    o_ref[...] = h @ wo_ref[...] + bo_ref[...][None, :]


def _tc_fin(s, h_prev, cnt_pad, bias, wr, wo, bo):
    return pl.pallas_call(
        _tc_fin_body,
        grid=(N // B,),
        in_specs=[
            pl.BlockSpec((2, B, 128), lambda b: (0, b, 0)),
            pl.BlockSpec((B, 256), lambda b: (b, 0)),
            pl.BlockSpec((2, B, 128), lambda b: (0, b, 0)),
            pl.BlockSpec((128,), lambda b: (0,)),
            pl.BlockSpec((256, 128), lambda b: (0, 0)),
            pl.BlockSpec((128, 64), lambda b: (0, 0)),
            pl.BlockSpec((64,), lambda b: (0,)),
        ],
        out_specs=pl.BlockSpec((B, 64), lambda b: (b, 0)),
        out_shape=jax.ShapeDtypeStruct((N, 64), jnp.float32),
    )(s, h_prev, cnt_pad, bias, wr, wo, bo)


def kernel(x, edge_index, Wl1, Wr1, b1, Wl2, Wr2, b2, Wl3, Wr3, b3, Wo, bo):
    src = edge_index[0].astype(jnp.int32)
    dst = edge_index[1].astype(jnp.int32)

    C_CS, C_ES = 80, 40
    ep = NS * C_CS * CHUNK
    pad = ep - E
    src_p = jnp.concatenate([src, jnp.zeros((pad,), jnp.int32)])
    dst_p = jnp.concatenate([dst, jnp.full((pad,), TRASH, jnp.int32)])

    src_cs = jnp.stack([src_p, src_p + N]).reshape(2, NS, C_CS, CHUNK)
    dst_cs = jnp.broadcast_to(dst_p.reshape(1, NS, C_CS, CHUNK),
                              (2, NS, C_CS, CHUNK))
    src_es = src_p.reshape(2, NS, C_ES, CHUNK)
    dst_es = dst_p.reshape(2, NS, C_ES, CHUNK)

    seg_cs = _sc_segsum(C_CS, 128)
    seg_es = _sc_segsum(C_ES, 128)

    y1 = _tc_pre(x, Wl1).reshape(2 * N, 128)
    cnt2 = _sc_counts(C_ES)(dst_es)
    s1 = seg_cs(y1, src_cs, dst_cs)
    h1, y2 = _tc_mid(s1, x, cnt2, b1, Wr1, Wl2, split_next=True)
    s2 = seg_cs(y2.reshape(2 * N, 128), src_cs, dst_cs)
    h2, y3 = _tc_mid(s2, h1, cnt2, b2, Wr2, Wl3, split_next=False)
    s3 = seg_es(y3, src_es, dst_es)
    return _tc_fin(s3, h2, cnt2, b3, Wr3, Wo, bo)


# speedup vs baseline: 1.0075x; 1.0075x over previous
import functools

import jax
import jax.numpy as jnp
from jax import lax
from jax.experimental import pallas as pl
from jax.experimental.pallas import tpu as pltpu
from jax.experimental.pallas import tpu_sc as plsc

N = 10000
E = 160000
NPAD = 10240
TRASH = N
NC, NS = 2, 16
CHUNK = 128
SEC = 8
B = 5000


def _sc_segsum(C, D):
    mesh = plsc.VectorSubcoreMesh(core_axis_name="c", subcore_axis_name="s",
                                   num_cores=NC, num_subcores=NS)
    zrows = NPAD // NS
    nsec = C // SEC
    psec = SEC // 2
    assert C % SEC == 0

    @functools.partial(
        pl.kernel,
        out_type=jax.ShapeDtypeStruct((2, NPAD, D), jnp.float32),
        mesh=mesh,
        scratch_types=[
            pltpu.VMEM((2, SEC, CHUNK), jnp.int32),
            pltpu.VMEM((2, SEC, CHUNK), jnp.int32),
            pltpu.VMEM((CHUNK, D), jnp.float32),
            pltpu.VMEM((CHUNK, D), jnp.float32),
            pltpu.VMEM((16, D), jnp.float32),
            pltpu.VMEM_SHARED((NPAD, D), jnp.float32),
            pltpu.SemaphoreType.DMA,
            pltpu.SemaphoreType.DMA,
            pltpu.SemaphoreType.DMA,
            pltpu.SemaphoreType.DMA,
            pltpu.SemaphoreType.DMA,
        ],
    )
    def k(table, src_i, dst_i, out, src_s, dst_s, g0, g1, zbuf, acc,
          sg0, sg1, ss0, ss1, sz):
        c = lax.axis_index("c")
        s = lax.axis_index("s")

        for r in range(16):
            for q in range(D // 16):
                zbuf[r, pl.ds(q * 16, 16)] = jnp.zeros((16,), jnp.float32)

        def zbody(i, carry):
            pltpu.async_copy(zbuf, acc.at[pl.ds(s * zrows + i * 16, 16)], sz)
            return carry
        lax.fori_loop(0, zrows // 16, zbody, 0)
        pltpu.sync_copy(src_i.at[c, s, pl.ds(0, SEC)], src_s.at[0])
        pltpu.sync_copy(dst_i.at[c, s, pl.ds(0, SEC)], dst_s.at[0])

        def zdrain(i, carry):
            pltpu.make_async_copy(zbuf, acc.at[pl.ds(s * zrows, 16)], sz).wait()
            return carry
        lax.fori_loop(0, zrows // 16, zdrain, 0)
        plsc.subcore_barrier()

        def src_sl(x):
            return src_s.at[(x // SEC) % 2, x % SEC]

        def dst_sl(x):
            return dst_s.at[(x // SEC) % 2, x % SEC]

        def wait_g(buf, sem, x):
            pltpu.make_async_copy(table.at[src_sl(x)], buf, sem).wait()

        def wait_s(buf, sem, x):
            pltpu.make_async_copy(buf, acc.at[dst_sl(x)], sem).wait()

        pltpu.async_copy(table.at[src_sl(0)], g0, sg0)

        def mbody(i, carry):
            a = 2 * i
            b = a + 1
            sec = i // psec
            wait_g(g0, sg0, a)
            pltpu.async_copy(g0, acc.at[dst_sl(a)], ss0, add=True)

            @pl.when(i > 0)
            def _():
                wait_s(g1, ss1, a)

            @pl.when((i % psec == 0) & (sec + 1 < nsec))
            def _():
                pltpu.async_copy(src_i.at[c, s, pl.ds((sec + 1) * SEC, SEC)],
                                 src_s.at[(sec + 1) % 2], sz)
                pltpu.async_copy(dst_i.at[c, s, pl.ds((sec + 1) * SEC, SEC)],
                                 dst_s.at[(sec + 1) % 2], sz)

            pltpu.async_copy(table.at[src_sl(b)], g1, sg1)
            wait_g(g1, sg1, b)
            pltpu.async_copy(g1, acc.at[dst_sl(b)], ss1, add=True)
            wait_s(g0, ss0, a)

            @pl.when(((i + 1) % psec == 0) & (sec + 1 < nsec))
            def _():
                pltpu.make_async_copy(src_i.at[c, s, pl.ds(0, SEC)],
                                      src_s.at[0], sz).wait()
                pltpu.make_async_copy(dst_i.at[c, s, pl.ds(0, SEC)],
                                      dst_s.at[0], sz).wait()

            @pl.when(b + 1 < C)
            def _():
                pltpu.async_copy(table.at[src_sl(b + 1)], g0, sg0)
            return carry
        lax.fori_loop(0, C // 2, mbody, 0)

        wait_s(g1, ss1, C - 1)
        plsc.subcore_barrier()
        pltpu.sync_copy(acc.at[pl.ds(s * zrows, zrows)],
                        out.at[c, pl.ds(s * zrows, zrows)])

    return k


def _sc_counts(C):
    D = 128
    mesh = plsc.VectorSubcoreMesh(core_axis_name="c", subcore_axis_name="s",
                                   num_cores=NC, num_subcores=NS)
    zrows = NPAD // NS

    @functools.partial(
        pl.kernel,
        out_type=jax.ShapeDtypeStruct((2, NPAD, D), jnp.float32),
        mesh=mesh,
        scratch_types=[
            pltpu.VMEM((C, CHUNK), jnp.int32),
            pltpu.VMEM((CHUNK, D), jnp.float32),
            pltpu.VMEM((16, D), jnp.float32),
            pltpu.VMEM_SHARED((NPAD, D), jnp.float32),
            pltpu.SemaphoreType.DMA,
        ],
    )
    def k(dst_i, out, dst_v, ones_b, zbuf, acc, sz):
        c = lax.axis_index("c")
        s = lax.axis_index("s")

        for r in range(16):
            for q in range(D // 16):
                zbuf[r, pl.ds(q * 16, 16)] = jnp.zeros((16,), jnp.float32)
        for r in range(CHUNK):
            for q in range(D // 16):
                ones_b[r, pl.ds(q * 16, 16)] = jnp.ones((16,), jnp.float32)

        def zbody(i, carry):
            pltpu.async_copy(zbuf, acc.at[pl.ds(s * zrows + i * 16, 16)], sz)
            return carry
        lax.fori_loop(0, zrows // 16, zbody, 0)

        pltpu.sync_copy(dst_i.at[c, s], dst_v)

        def zdrain(i, carry):
            pltpu.make_async_copy(zbuf, acc.at[pl.ds(s * zrows, 16)], sz).wait()
            return carry
        lax.fori_loop(0, zrows // 16, zdrain, 0)
        plsc.subcore_barrier()

        def mbody(j, carry):
            pltpu.sync_copy(ones_b, acc.at[dst_v.at[j]], add=True)
            return carry
        lax.fori_loop(0, C, mbody, 0)

        plsc.subcore_barrier()
        pltpu.sync_copy(acc.at[pl.ds(s * zrows, zrows)],
                        out.at[c, pl.ds(s * zrows, zrows)])

    return k


def _tc_pre_body(x_ref, wl_ref, y_ref):
    y = x_ref[...] @ wl_ref[...]
    y_ref[0] = y[:, :128]
    y_ref[1] = y[:, 128:]


def _tc_pre(x, wl):
    return pl.pallas_call(
        _tc_pre_body,
        grid=(N // B,),
        in_specs=[
            pl.BlockSpec((B, 256), lambda b: (b, 0)),
            pl.BlockSpec((256, 256), lambda b: (0, 0)),
        ],
        out_specs=pl.BlockSpec((2, B, 128), lambda b: (0, b, 0)),
        out_shape=jax.ShapeDtypeStruct((2, N, 128), jnp.float32),
    )(x, wl)


def _mean_root(s_ref, h_ref, cnt_ref, bias_ref, wr_ref, split):
    cnt = cnt_ref[0, :, 0:1] + cnt_ref[1, :, 0:1]
    inv = 1.0 / jnp.maximum(cnt, 1.0)
    if split:
        ssum = jnp.concatenate([s_ref[0], s_ref[1]], axis=1)
    else:
        ssum = s_ref[0] + s_ref[1]
    pre = ssum * inv + bias_ref[...][None, :] + h_ref[...] @ wr_ref[...]
    return jnp.maximum(pre, 0.0)


def _tc_mid_body(split_next, s_ref, h_ref, cnt_ref, bias_ref, wr_ref, wln_ref,
                 hout_ref, y_ref):
    h = _mean_root(s_ref, h_ref, cnt_ref, bias_ref, wr_ref, split=True)
    hout_ref[...] = h
    y = h @ wln_ref[...]
    if split_next:
        y_ref[0] = y[:, :128]
        y_ref[1] = y[:, 128:]
    else:
        y_ref[...] = y


def _tc_mid(s, h_prev, cnt_pad, bias, wr, wl_next, split_next):
    d_next = wl_next.shape[1]
    if split_next:
        y_shape = jax.ShapeDtypeStruct((2, N, 128), jnp.float32)
        y_spec = pl.BlockSpec((2, B, 128), lambda b: (0, b, 0))
    else:
        y_shape = jax.ShapeDtypeStruct((N, d_next), jnp.float32)
        y_spec = pl.BlockSpec((B, d_next), lambda b: (b, 0))
    return pl.pallas_call(
        functools.partial(_tc_mid_body, split_next),
        grid=(N // B,),
        in_specs=[
            pl.BlockSpec((2, B, 128), lambda b: (0, b, 0)),
            pl.BlockSpec((B, 256), lambda b: (b, 0)),
            pl.BlockSpec((2, B, 128), lambda b: (0, b, 0)),
            pl.BlockSpec((256,), lambda b: (0,)),
            pl.BlockSpec((256, 256), lambda b: (0, 0)),
            pl.BlockSpec((256, d_next), lambda b: (0, 0)),
        ],
        out_specs=[
            pl.BlockSpec((B, 256), lambda b: (b, 0)),
            y_spec,
        ],
        out_shape=[
            jax.ShapeDtypeStruct((N, 256), jnp.float32),
            y_shape,
        ],
    )(s, h_prev, cnt_pad, bias, wr, wl_next)


def _tc_fin_body(s_ref, h_ref, cnt_ref, bias_ref, wr_ref, wo_ref, bo_ref,
                 o_ref):
    h = _mean_root(s_ref, h_ref, cnt_ref, bias_ref, wr_ref, split=False)
    o_ref[...] = h @ wo_ref[...] + bo_ref[...][None, :]


def _tc_fin(s, h_prev, cnt_pad, bias, wr, wo, bo):
    return pl.pallas_call(
        _tc_fin_body,
        grid=(N // B,),
        in_specs=[
            pl.BlockSpec((2, B, 128), lambda b: (0, b, 0)),
            pl.BlockSpec((B, 256), lambda b: (b, 0)),
            pl.BlockSpec((2, B, 128), lambda b: (0, b, 0)),
            pl.BlockSpec((128,), lambda b: (0,)),
            pl.BlockSpec((256, 128), lambda b: (0, 0)),
            pl.BlockSpec((128, 64), lambda b: (0, 0)),
            pl.BlockSpec((64,), lambda b: (0,)),
        ],
        out_specs=pl.BlockSpec((B, 64), lambda b: (b, 0)),
        out_shape=jax.ShapeDtypeStruct((N, 64), jnp.float32),
    )(s, h_prev, cnt_pad, bias, wr, wo, bo)


def kernel(x, edge_index, Wl1, Wr1, b1, Wl2, Wr2, b2, Wl3, Wr3, b3, Wo, bo):
    src = edge_index[0].astype(jnp.int32)
    dst = edge_index[1].astype(jnp.int32)

    C_CS, C_ES = 80, 40
    ep = NS * C_CS * CHUNK
    pad = ep - E
    src_p = jnp.concatenate([src, jnp.zeros((pad,), jnp.int32)])
    dst_p = jnp.concatenate([dst, jnp.full((pad,), TRASH, jnp.int32)])

    src_cs = jnp.stack([src_p, src_p + N]).reshape(2, NS, C_CS, CHUNK)
    dst_cs = jnp.broadcast_to(dst_p.reshape(1, NS, C_CS, CHUNK),
                              (2, NS, C_CS, CHUNK))
    src_es = src_p.reshape(2, NS, C_ES, CHUNK)
    dst_es = dst_p.reshape(2, NS, C_ES, CHUNK)

    seg_cs = _sc_segsum(C_CS, 128)
    seg_es = _sc_segsum(C_ES, 128)

    y1 = _tc_pre(x, Wl1).reshape(2 * N, 128)
    cnt2 = _sc_counts(C_ES)(dst_es)
    s1 = seg_cs(y1, src_cs, dst_cs)
    h1, y2 = _tc_mid(s1, x, cnt2, b1, Wr1, Wl2, split_next=True)
    s2 = seg_cs(y2.reshape(2 * N, 128), src_cs, dst_cs)
    h2, y3 = _tc_mid(s2, h1, cnt2, b2, Wr2, Wl3, split_next=False)
    s3 = seg_es(y3, src_es, dst_es)
    return _tc_fin(s3, h2, cnt2, b3, Wr3, Wo, bo)
